# baseline (speedup 1.0000x reference)
import functools

import jax
import jax.numpy as jnp
from jax import lax
from jax.experimental import pallas as pl
from jax.experimental.pallas import tpu as pltpu
from jax.experimental.pallas import tpu_sc as plsc

H = W = 14
HW = H * W
NROWS = HW * HW
CHUNK = 112
NVREG = CHUNK // 16
NCHUNK = NROWS // CHUNK


def kernel(unique_params, index_map):
    info = plsc.get_sparse_core_info()
    nc, ns = info.num_cores, info.num_subcores
    nw = nc * ns
    trips = -(-NCHUNK // nw)
    if trips % 2:
        trips += 1

    mesh = plsc.VectorSubcoreMesh(core_axis_name="c", subcore_axis_name="s")

    @functools.partial(
        pl.kernel,
        mesh=mesh,
        out_type=jax.ShapeDtypeStruct((NROWS, 32, 32), jnp.float32),
        scratch_types=[
            pltpu.VMEM((CHUNK,), jnp.int32),
            pltpu.VMEM((CHUNK,), jnp.int32),
            pltpu.SemaphoreType.DMA,
            pltpu.SemaphoreType.DMA,
        ],
    )
    def gather_rows(table_hbm, idx_hbm, out_hbm, idx_v0, idx_v1, sem0, sem1):
        wid = lax.axis_index("s") * nc + lax.axis_index("c")
        idx_v = (idx_v0, idx_v1)
        sem = (sem0, sem1)

        def start(t, b):
            c = wid + nw * t

            @pl.when(c < NCHUNK)
            def _():
                base = c * CHUNK
                pltpu.sync_copy(idx_hbm.at[pl.ds(base, CHUNK)], idx_v[b])

                def per_vreg(q, carry):
                    v16 = idx_v[b][pl.ds(q * 16, 16)]
                    r0 = base + q * 16
                    for l in range(16):
                        pltpu.async_copy(table_hbm.at[v16[l]],
                                         out_hbm.at[r0 + l], sem[b])
                    return carry

                lax.fori_loop(0, NVREG, per_vreg, None)

        def finish(t, b):
            c = wid + nw * t

            @pl.when(c < NCHUNK)
            def _():
                pltpu.make_async_copy(
                    table_hbm.at[pl.ds(0, CHUNK)],
                    out_hbm.at[pl.ds(c * CHUNK, CHUNK)], sem[b]).wait()

        start(0, 0)

        def body(u, carry):
            t0 = 2 * u
            start(t0 + 1, 1)
            finish(t0, 0)
            start(t0 + 2, 0)
            finish(t0 + 1, 1)
            return carry

        lax.fori_loop(0, trips // 2, body, None)

    out = gather_rows(unique_params, index_map.reshape(NROWS))
    return out.reshape(HW, HW, 32, 32)


# speedup vs baseline: 58.6814x; 58.6814x over previous
import functools

import jax
import jax.numpy as jnp
from jax import lax
from jax.experimental import pallas as pl
from jax.experimental.pallas import tpu as pltpu
from jax.experimental.pallas import tpu_sc as plsc

H = W = 14
HW = H * W
CHUNK = 56
TAILC = 32
MAINT = 24
TAIL = 4
PERN = 4
STRIDE = PERN * CHUNK
NCHUNK = HW * PERN


def kernel(unique_params, index_map):
    info = plsc.get_sparse_core_info()
    nc, ns = info.num_cores, info.num_subcores
    nw = nc * ns
    trips = -(-NCHUNK // nw)
    if trips % 2:
        trips += 1

    mesh = plsc.VectorSubcoreMesh(core_axis_name="c", subcore_axis_name="s")

    @functools.partial(
        pl.kernel,
        mesh=mesh,
        out_type=(jax.ShapeDtypeStruct((HW, HW, 1024), jnp.float32),
                  jax.ShapeDtypeStruct((HW, 8, 1024), jnp.float32)),
        scratch_types=[
            pltpu.VMEM((CHUNK,), jnp.int32),
            pltpu.VMEM((CHUNK,), jnp.int32),
            pltpu.VMEM((CHUNK, 1024), jnp.float32),
            pltpu.VMEM((CHUNK, 1024), jnp.float32),
            pltpu.SemaphoreType.DMA,
            pltpu.SemaphoreType.DMA,
        ],
    )
    def gather_rows(table_hbm, idx_hbm, out_hbm, tail_hbm,
                    idx_v0, idx_v1, rows_v0, rows_v1, sem0, sem1):
        wid = lax.axis_index("s") * nc + lax.axis_index("c")
        idx_v = (idx_v0, idx_v1)
        rows_v = (rows_v0, rows_v1)
        sem = (sem0, sem1)

        def start(t, b):
            c = wid + nw * t

            @pl.when(c < NCHUNK)
            def _():
                k = c % PERN
                base = (c // PERN) * STRIDE + k * CHUNK

                @pl.when(k < PERN - 1)
                def _():
                    pltpu.sync_copy(idx_hbm.at[pl.ds(base, CHUNK)], idx_v[b])
                    pltpu.async_copy(table_hbm.at[idx_v[b]], rows_v[b], sem[b])

                @pl.when(k == PERN - 1)
                def _():
                    pltpu.sync_copy(idx_hbm.at[pl.ds(base, TAILC)],
                                    idx_v[b].at[pl.ds(0, TAILC)])
                    pltpu.async_copy(table_hbm.at[idx_v[b].at[pl.ds(0, TAILC)]],
                                     rows_v[b].at[pl.ds(0, TAILC)], sem[b])

        def finish(t, b):
            c = wid + nw * t

            @pl.when(c < NCHUNK)
            def _():
                n = c // PERN
                k = c % PERN
                j0 = k * CHUNK

                @pl.when(k < PERN - 1)
                def _():
                    pltpu.make_async_copy(table_hbm.at[idx_v[b]],
                                          rows_v[b], sem[b]).wait()
                    pltpu.sync_copy(rows_v[b], out_hbm.at[n, pl.ds(j0, CHUNK)])

                @pl.when(k == PERN - 1)
                def _():
                    pltpu.make_async_copy(
                        table_hbm.at[idx_v[b].at[pl.ds(0, TAILC)]],
                        rows_v[b].at[pl.ds(0, TAILC)], sem[b]).wait()
                    pltpu.sync_copy(rows_v[b].at[pl.ds(0, MAINT)],
                                    out_hbm.at[n, pl.ds(j0, MAINT)])
                    pltpu.sync_copy(rows_v[b].at[pl.ds(MAINT, 8)],
                                    tail_hbm.at[n])

        start(0, 0)

        def body(u, carry):
            t0 = 2 * u
            start(t0 + 1, 1)
            finish(t0, 0)
            start(t0 + 2, 0)
            finish(t0 + 1, 1)
            return carry

        lax.fori_loop(0, trips // 2, body, None)

    idxp = jnp.pad(index_map, ((0, 0), (0, STRIDE - HW))).reshape(HW * STRIDE)
    main, tails = gather_rows(unique_params.reshape(729, 1024), idxp)
    out = lax.dynamic_update_slice(main, tails[:, :TAIL, :], (0, HW - TAIL, 0))
    return out.reshape(HW, HW, 32, 32)
